# baseline (speedup 1.0000x reference)
import jax
import jax.numpy as jnp
from jax import lax
from jax.experimental import pallas as pl
from jax.experimental.pallas import tpu as pltpu
from jax.experimental.pallas import tpu_sc as plsc

_B = 128
_V = 100000
_MARGIN = 1.0

_VS = 64000
_SPW = _VS // 32
_CV = 200
_NCHK = _SPW // _CV
_G = 8
_NBUF = 2

_BV = 800
_TCN = (_V - _VS) // _BV
_TCK = 8

_NEG = float("-inf")


def _body(xt_hbm, gold_hbm, out_hbm, buf0, buf1, gold_v, stage_v,
          allbuf, grow_v, out_v, shared, sem0, sem1, gsem):
    c = lax.axis_index("c")
    s = lax.axis_index("s")
    w = c * 16 + s
    iota = lax.iota(jnp.int32, 16)
    bufs = (buf0, buf1)
    sems = (sem0, sem1)

    wbase = w * _SPW

    def start(k):
        voff = pl.multiple_of(wbase + k * _CV, 8)
        return pltpu.async_copy(xt_hbm.at[pl.ds(voff, _CV), :],
                                bufs[k % _NBUF], sems[k % _NBUF])

    cps = [None] * _NCHK
    for k in range(_NBUF):
        cps[k] = start(k)
    pltpu.sync_copy(gold_hbm, gold_v)

    def mk_body(buf):
        def body(i, carry):
            accs = list(carry)
            for j in range(_G):
                vloc = i * _G + j
                for seg in range(8):
                    v = buf[vloc, pl.ds(seg * 16, 16)]
                    a1, a2 = accs[2 * seg], accs[2 * seg + 1]
                    accs[2 * seg + 1] = jnp.maximum(a2, jnp.minimum(a1, v))
                    accs[2 * seg] = jnp.maximum(a1, v)
            return tuple(accs)
        return body

    acc = [jnp.full((16,), _NEG, dtype=jnp.float32)] * 16
    for k in range(_NCHK):
        cps[k].wait()
        acc = list(lax.fori_loop(0, _CV // _G, mk_body(bufs[k % _NBUF]),
                                 tuple(acc)))
        if k + _NBUF < _NCHK:
            cps[k + _NBUF] = start(k + _NBUF)

    for seg in range(8):
        stage_v[pl.ds(seg * 32, 16)] = acc[2 * seg]
        stage_v[pl.ds(seg * 32 + 16, 16)] = acc[2 * seg + 1]
    pltpu.sync_copy(stage_v, shared.at[pl.ds(s * 256, 256)])
    plsc.subcore_barrier()

    @pl.when(s < 8)
    def _merge():
        pltpu.sync_copy(shared, allbuf)
        soff = s * 32
        m1 = jnp.full((16,), _NEG, dtype=jnp.float32)
        m2 = jnp.full((16,), _NEG, dtype=jnp.float32)
        for j in range(16):
            a1 = allbuf[pl.ds(j * 256 + soff, 16)]
            a2 = allbuf[pl.ds(j * 256 + soff + 16, 16)]
            m2 = jnp.maximum(jnp.maximum(m2, a2), jnp.minimum(m1, a1))
            m1 = jnp.maximum(m1, a1)
        idxv = gold_v[pl.ds(s * 16, 16)]
        pltpu.async_copy(xt_hbm.at[idxv], grow_v, gsem).wait()
        g = jnp.full((16,), _NEG, dtype=jnp.float32)
        for l in range(16):
            rowv = grow_v[l, pl.ds(s * 16, 16)]
            g = jnp.where(iota == l, rowv, g)
        out_v[pl.ds(0, 16)] = m1
        out_v[pl.ds(16, 16)] = m2
        out_v[pl.ds(32, 16)] = g
        obase = pl.multiple_of((c * 8 + s) * 128, 128)
        pltpu.sync_copy(out_v, out_hbm.at[pl.ds(obase, 48)])


_sc_call = pl.kernel(
    _body,
    name="ranking_loss_sc",
    out_type=jax.ShapeDtypeStruct((16 * 128,), jnp.float32),
    mesh=plsc.VectorSubcoreMesh(core_axis_name="c", subcore_axis_name="s"),
    compiler_params=pltpu.CompilerParams(needs_layout_passes=False,
                                         use_tc_tiling_on_sc=True),
    scratch_types=[
        pltpu.VMEM((_CV, _B), jnp.float32),
        pltpu.VMEM((_CV, _B), jnp.float32),
        pltpu.VMEM((_B,), jnp.int32),
        pltpu.VMEM((256,), jnp.float32),
        pltpu.VMEM((4096,), jnp.float32),
        pltpu.VMEM((16, _B), jnp.float32),
        pltpu.VMEM((48,), jnp.float32),
        pltpu.VMEM_SHARED((4096,), jnp.float32),
        pltpu.SemaphoreType.DMA,
        pltpu.SemaphoreType.DMA,
        pltpu.SemaphoreType.DMA,
    ],
)


def _tc_body(xt_ref, o_ref, acc_ref):
    pid = pl.program_id(0)

    @pl.when(pid == 0)
    def _init():
        acc_ref[...] = jnp.full((2 * _TCK, 8, _B), _NEG, dtype=jnp.float32)

    accs = [acc_ref[q] for q in range(2 * _TCK)]
    blk = xt_ref[...]
    for t in range(_BV // 8):
        q = t % _TCK
        v = blk[8 * t:8 * (t + 1), :]
        a1, a2 = accs[2 * q], accs[2 * q + 1]
        accs[2 * q + 1] = jnp.maximum(a2, jnp.minimum(a1, v))
        accs[2 * q] = jnp.maximum(a1, v)
    for q in range(2 * _TCK):
        acc_ref[q] = accs[q]

    @pl.when(pid == _TCN - 1)
    def _fold():
        m1, m2 = accs[0], accs[1]
        for q in range(1, _TCK):
            b1, b2 = accs[2 * q], accs[2 * q + 1]
            m2 = jnp.maximum(jnp.maximum(m2, b2), jnp.minimum(m1, b1))
            m1 = jnp.maximum(m1, b1)
        o_ref[0] = m1
        o_ref[1] = m2


_tc_call = pl.pallas_call(
    _tc_body,
    grid=(_TCN,),
    in_specs=[pl.BlockSpec((_BV, _B), lambda i: (i + _VS // _BV, 0))],
    out_specs=pl.BlockSpec((2, 8, _B), lambda i: (0, 0, 0)),
    out_shape=jax.ShapeDtypeStruct((2, 8, _B), jnp.float32),
    scratch_shapes=[pltpu.VMEM((2 * _TCK, 8, _B), jnp.float32)],
)


def _merge_pairs(a1, a2, b1, b2):
    m1 = jnp.maximum(a1, b1)
    m2 = jnp.maximum(jnp.maximum(a2, b2), jnp.minimum(a1, b1))
    return m1, m2


@jax.jit
def kernel(x, gold):
    xt = x.T
    sc = _sc_call(xt, gold)
    tc = _tc_call(xt)
    p = sc.reshape(16, 128)[:, :48].reshape(2, 8, 3, 16)
    a, b = p[0], p[1]
    m1, m2 = _merge_pairs(a[:, 0].reshape(-1), a[:, 1].reshape(-1),
                          b[:, 0].reshape(-1), b[:, 1].reshape(-1))
    g = jnp.maximum(a[:, 2], b[:, 2]).reshape(-1)
    t1s, t2s = tc[0], tc[1]
    t1 = jnp.max(t1s, axis=0)
    am = jnp.argmax(t1s, axis=0)
    masked = jnp.where(jnp.arange(8)[:, None] == am[None, :],
                       jnp.float32(_NEG), t1s)
    t2 = jnp.maximum(jnp.max(masked, axis=0), jnp.max(t2s, axis=0))
    m1, m2 = _merge_pairs(m1, m2, t1, t2)
    loss = jnp.where(g == m1,
                     jnp.maximum(jnp.float32(_MARGIN) + m2 - m1, 0.0),
                     jnp.float32(_MARGIN) + m1 - g)
    return jnp.sum(loss) / jnp.float32(_B)


# speedup vs baseline: 1.0480x; 1.0480x over previous
import jax
import jax.numpy as jnp
from jax import lax
from jax.experimental import pallas as pl
from jax.experimental.pallas import tpu as pltpu
from jax.experimental.pallas import tpu_sc as plsc

_B = 128
_V = 100000
_MARGIN = 1.0

_TPW = 391
_KT = 23
_NCHK = 17
_CV = _KT * 8
_SPW = _TPW * 8
_VLAST = _V - _CV
_G = 8
_NBUF = 3

_NEG = float("-inf")
_POS = float("inf")


def _body(xt_hbm, gold_hbm, out_hbm, buf0, buf1, buf2, gold_v, stage_v,
          allbuf, grow_v, out_v, shared, sem0, sem1, sem2, gsem):
    c = lax.axis_index("c")
    s = lax.axis_index("s")
    w = c * 16 + s
    iota = lax.iota(jnp.int32, 16)
    bufs = (buf0, buf1, buf2)
    sems = (sem0, sem1, sem2)

    wbase = w * _SPW

    def start(k):
        voff_u = wbase + k * _CV
        voff = jnp.minimum(voff_u, _VLAST) if k == _NCHK - 1 else voff_u
        voff = pl.multiple_of(voff, 8)
        return pltpu.async_copy(xt_hbm.at[pl.ds(voff, _CV), :],
                                bufs[k % _NBUF], sems[k % _NBUF])

    cps = [None] * _NCHK
    for k in range(_NBUF):
        cps[k] = start(k)
    pltpu.sync_copy(gold_hbm, gold_v)

    voff_u_last = wbase + (_NCHK - 1) * _CV
    thresh = voff_u_last - jnp.minimum(voff_u_last, _VLAST)

    def mk_body(buf, last):
        def body(i, carry):
            accs = list(carry)
            for j in range(_G):
                vloc = i * _G + j
                if last:
                    pen = jnp.where(vloc >= thresh,
                                    jnp.float32(_POS), jnp.float32(_NEG))
                for seg in range(8):
                    v = buf[vloc, pl.ds(seg * 16, 16)]
                    if last:
                        v = jnp.minimum(v, pen)
                    a1, a2 = accs[2 * seg], accs[2 * seg + 1]
                    accs[2 * seg + 1] = jnp.maximum(a2, jnp.minimum(a1, v))
                    accs[2 * seg] = jnp.maximum(a1, v)
            return tuple(accs)
        return body

    acc = [jnp.full((16,), _NEG, dtype=jnp.float32)] * 16
    for k in range(_NCHK):
        cps[k].wait()
        acc = list(lax.fori_loop(0, _CV // _G,
                                 mk_body(bufs[k % _NBUF], k == _NCHK - 1),
                                 tuple(acc)))
        if k + _NBUF < _NCHK:
            cps[k + _NBUF] = start(k + _NBUF)

    for seg in range(8):
        stage_v[pl.ds(seg * 32, 16)] = acc[2 * seg]
        stage_v[pl.ds(seg * 32 + 16, 16)] = acc[2 * seg + 1]
    pltpu.sync_copy(stage_v, shared.at[pl.ds(s * 256, 256)])
    plsc.subcore_barrier()

    @pl.when(s < 8)
    def _merge():
        pltpu.sync_copy(shared, allbuf)
        soff = s * 32
        m1 = jnp.full((16,), _NEG, dtype=jnp.float32)
        m2 = jnp.full((16,), _NEG, dtype=jnp.float32)
        for j in range(16):
            a1 = allbuf[pl.ds(j * 256 + soff, 16)]
            a2 = allbuf[pl.ds(j * 256 + soff + 16, 16)]
            m2 = jnp.maximum(jnp.maximum(m2, a2), jnp.minimum(m1, a1))
            m1 = jnp.maximum(m1, a1)
        idxv = gold_v[pl.ds(s * 16, 16)]
        pltpu.async_copy(xt_hbm.at[idxv], grow_v, gsem).wait()
        g = jnp.full((16,), _NEG, dtype=jnp.float32)
        for l in range(16):
            rowv = grow_v[l, pl.ds(s * 16, 16)]
            g = jnp.where(iota == l, rowv, g)
        out_v[pl.ds(0, 16)] = m1
        out_v[pl.ds(16, 16)] = m2
        out_v[pl.ds(32, 16)] = g
        obase = pl.multiple_of((c * 8 + s) * 128, 128)
        pltpu.sync_copy(out_v, out_hbm.at[pl.ds(obase, 48)])


_sc_call = pl.kernel(
    _body,
    name="ranking_loss_sc",
    out_type=jax.ShapeDtypeStruct((16 * 128,), jnp.float32),
    mesh=plsc.VectorSubcoreMesh(core_axis_name="c", subcore_axis_name="s"),
    compiler_params=pltpu.CompilerParams(needs_layout_passes=False,
                                         use_tc_tiling_on_sc=True),
    scratch_types=[
        pltpu.VMEM((_CV, _B), jnp.float32),
        pltpu.VMEM((_CV, _B), jnp.float32),
        pltpu.VMEM((_CV, _B), jnp.float32),
        pltpu.VMEM((_B,), jnp.int32),
        pltpu.VMEM((256,), jnp.float32),
        pltpu.VMEM((4096,), jnp.float32),
        pltpu.VMEM((16, _B), jnp.float32),
        pltpu.VMEM((48,), jnp.float32),
        pltpu.VMEM_SHARED((4096,), jnp.float32),
        pltpu.SemaphoreType.DMA,
        pltpu.SemaphoreType.DMA,
        pltpu.SemaphoreType.DMA,
        pltpu.SemaphoreType.DMA,
    ],
)


@jax.jit
def kernel(x, gold):
    partials = _sc_call(x.T, gold)
    p = partials.reshape(16, 128)[:, :48].reshape(2, 8, 3, 16)
    a, b = p[0], p[1]
    m1 = jnp.maximum(a[:, 0], b[:, 0])
    m2 = jnp.maximum(jnp.maximum(a[:, 1], b[:, 1]),
                     jnp.minimum(a[:, 0], b[:, 0]))
    g = jnp.maximum(a[:, 2], b[:, 2])
    loss = jnp.where(g == m1,
                     jnp.maximum(jnp.float32(_MARGIN) + m2 - m1, 0.0),
                     jnp.float32(_MARGIN) + m1 - g)
    return jnp.sum(loss) / jnp.float32(_B)
